# baseline (speedup 1.0000x reference)
import functools

import jax
import jax.numpy as jnp
from jax import lax
from jax.experimental import pallas as pl


_ROWS = 16384
_COLS = 50
_CLASSES = 256
_BLOCK_R = 128


def _onehot_body(x_ref, out_ref):
    x = x_ref[...].astype(jnp.int32)
    classes = lax.broadcasted_iota(jnp.int32, (_BLOCK_R, _COLS, _CLASSES), 2)
    out_ref[...] = (x[:, :, None] == classes).astype(jnp.float32)


def kernel(x):
    grid = (_ROWS // _BLOCK_R,)
    return pl.pallas_call(
        _onehot_body,
        grid=grid,
        in_specs=[pl.BlockSpec((_BLOCK_R, _COLS), lambda i: (i, 0))],
        out_specs=pl.BlockSpec((_BLOCK_R, _COLS, _CLASSES), lambda i: (i, 0, 0)),
        out_shape=jax.ShapeDtypeStruct((_ROWS, _COLS, _CLASSES), jnp.float32),
    )(x)
